# baseline (speedup 1.0000x reference)
import functools

import jax
import jax.numpy as jnp
from jax import lax
from jax.experimental import pallas as pl
from jax.experimental.pallas import tpu as pltpu
from jax.experimental.pallas import tpu_sc as plsc

VOCAB = 1_000_000
EMBED = 64
BATCH = 4096
SEQ = 200

PACK_C = 2048
PACK_NBLK = (VOCAB + PACK_C - 1) // PACK_C
PACK_ROWS = PACK_NBLK * PACK_C * EMBED // 128

NC = 2
NS = 16
L = 16
NW = NC * NS
ROWS_PER_W = BATCH // NW
CH0 = 104
CH1 = SEQ - CH0
NCHUNK = EMBED // L

_mesh = plsc.VectorSubcoreMesh(core_axis_name="c", subcore_axis_name="s")


@functools.partial(
    pl.kernel,
    out_type=jax.ShapeDtypeStruct((BATCH, EMBED), jnp.float32),
    mesh=_mesh,
    scratch_types=[
        pltpu.VMEM((ROWS_PER_W, 208), jnp.int32),
        pltpu.VMEM((4, SEQ, EMBED), jnp.float32),
        pltpu.VMEM((ROWS_PER_W, EMBED), jnp.float32),
        pltpu.SemaphoreType.DMA,
        pltpu.SemaphoreType.DMA,
        pltpu.SemaphoreType.DMA,
        pltpu.SemaphoreType.DMA,
    ],
    compiler_params=pltpu.CompilerParams(use_tc_tiling_on_sc=False),
)
def _pool_kernel(tok_hbm, table_hbm, out_hbm, tok_v, buf_v, out_v,
                 sem0, sem1, sem2, sem3):
    wid = lax.axis_index("s") * NC + lax.axis_index("c")
    base = wid * ROWS_PER_W
    pltpu.sync_copy(tok_hbm.at[pl.ds(base, ROWS_PER_W), :],
                    tok_v.at[:, pl.ds(0, SEQ)])

    def remap_body(i, carry):
        for c in range(208 // L):
            v = tok_v[i, pl.ds(c * L, L)]
            v = (
                ((v >> 11) << 11)
                + ((v & 1023) << 1)
                + ((v >> 10) & 1)
            )
            tok_v[i, pl.ds(c * L, L)] = v
        return carry

    lax.fori_loop(0, ROWS_PER_W, remap_body, 0)

    sems = (sem0, sem1, sem2, sem3)
    NBUF = 4

    def row_copies(i, p):
        return (
            pltpu.make_async_copy(
                table_hbm.at[tok_v.at[i, pl.ds(0, CH0)]],
                buf_v.at[p, pl.ds(0, CH0), :],
                sems[p]),
            pltpu.make_async_copy(
                table_hbm.at[tok_v.at[i, pl.ds(CH0, CH1)]],
                buf_v.at[p, pl.ds(CH0, CH1), :],
                sems[p]),
        )

    def start_row(i, p):
        for c in row_copies(i, p):
            c.start()

    def wait_row(i, p):
        for c in row_copies(i, p):
            c.wait()

    UNROLL = 8

    def accum_row(i, p):
        def body(r, accs):
            accs = list(accs)
            for rr in range(UNROLL):
                row = r * UNROLL + rr
                for c in range(NCHUNK):
                    accs[c] = accs[c] + buf_v[p, row, pl.ds(c * L, L)]
            return tuple(accs)

        accs = tuple(jnp.zeros((L,), jnp.float32) for _ in range(NCHUNK))
        accs = lax.fori_loop(0, SEQ // UNROLL, body, accs)
        for c in range(NCHUNK):
            out_v[i, pl.ds(c * L, L)] = accs[c]

    for p in range(NBUF - 1):
        start_row(p, p)

    def step(k, carry):
        i0 = k * NBUF
        for b in range(NBUF):
            i = i0 + b

            @pl.when(i + NBUF - 1 < ROWS_PER_W)
            def _():
                start_row(i + NBUF - 1, (b + NBUF - 1) % NBUF)

            wait_row(i, b)
            accum_row(i, b)
        return carry

    lax.fori_loop(0, ROWS_PER_W // NBUF, step, 0)
    pltpu.sync_copy(out_v, out_hbm.at[pl.ds(base, ROWS_PER_W), :])


def _pack_body(in_ref, eye_ref, o_ref):
    x = in_ref[...]
    t = lax.dot_general(
        x, eye_ref[...], (((0,), (0,)), ((), ())),
        preferred_element_type=jnp.float32,
        precision=lax.Precision.HIGHEST,
    )
    o_ref[...] = jnp.concatenate(
        [t[: PACK_C // 2, :], t[PACK_C // 2 :, :]], axis=1
    )


def _pack(tabT):
    return pl.pallas_call(
        _pack_body,
        grid=(PACK_NBLK,),
        in_specs=[
            pl.BlockSpec((EMBED, PACK_C), lambda j: (0, j)),
            pl.BlockSpec((EMBED, EMBED), lambda j: (0, 0)),
        ],
        out_specs=pl.BlockSpec((PACK_C // 2, 128), lambda j: (j, 0)),
        out_shape=jax.ShapeDtypeStruct((PACK_ROWS, 128), jnp.float32),
    )(tabT, jnp.eye(EMBED, dtype=jnp.float32))


def _mm_body(x_ref, w_ref, b_ref, o_ref):
    x = x_ref[...] * (1.0 / SEQ)
    o_ref[...] = (
        jnp.dot(x, w_ref[...], preferred_element_type=jnp.float32) + b_ref[...]
    )


def _matmul(pooled, W, b2d):
    return pl.pallas_call(
        _mm_body,
        out_shape=jax.ShapeDtypeStruct((BATCH, EMBED), jnp.float32),
    )(pooled, W, b2d)


def kernel(token_ids, embedding_table, W, b):
    tok = token_ids.astype(jnp.int32)
    packed = _pack(embedding_table.T)
    tab_rows = packed.reshape(2 * PACK_ROWS, EMBED)
    pooled = _pool_kernel(tok, tab_rows)
    return _matmul(pooled, W, b.reshape(1, EMBED))


# speedup vs baseline: 1.2417x; 1.2417x over previous
import functools

import jax
import jax.numpy as jnp
from jax import lax
from jax.experimental import pallas as pl
from jax.experimental.pallas import tpu as pltpu
from jax.experimental.pallas import tpu_sc as plsc

VOCAB = 1_000_000
EMBED = 64
BATCH = 4096
SEQ = 200

PACK_C = 2048
PACK_NBLK = (VOCAB + PACK_C - 1) // PACK_C
PACK_ROWS = PACK_NBLK * PACK_C * EMBED // 128

NC = 2
NS = 16
L = 16
NW = NC * NS
ROWS_PER_W = BATCH // NW
CH0 = 104
CH1 = SEQ - CH0
NCHUNK = EMBED // L

_mesh = plsc.VectorSubcoreMesh(core_axis_name="c", subcore_axis_name="s")


@functools.partial(
    pl.kernel,
    out_type=jax.ShapeDtypeStruct((BATCH, EMBED), jnp.float32),
    mesh=_mesh,
    scratch_types=[
        pltpu.VMEM((ROWS_PER_W, 208), jnp.int32),
        pltpu.VMEM((4, SEQ, EMBED), jnp.float32),
        pltpu.VMEM((ROWS_PER_W, EMBED), jnp.float32),
        pltpu.SemaphoreType.DMA,
        pltpu.SemaphoreType.DMA,
        pltpu.SemaphoreType.DMA,
        pltpu.SemaphoreType.DMA,
    ],
    compiler_params=pltpu.CompilerParams(use_tc_tiling_on_sc=False),
)
def _pool_kernel(tok_hbm, table_hbm, out_hbm, tok_v, buf_v, out_v,
                 sem0, sem1, sem2, sem3):
    wid = lax.axis_index("s") * NC + lax.axis_index("c")
    base = wid * ROWS_PER_W
    pltpu.sync_copy(tok_hbm.at[pl.ds(base, ROWS_PER_W), :],
                    tok_v.at[:, pl.ds(0, SEQ)])

    def remap_body(i, carry):
        for c in range(208 // L):
            v = tok_v[i, pl.ds(c * L, L)]
            v = (
                ((v >> 11) << 11)
                + ((v & 1023) << 1)
                + ((v >> 10) & 1)
            )
            tok_v[i, pl.ds(c * L, L)] = v
        return carry

    lax.fori_loop(0, ROWS_PER_W, remap_body, 0)

    sems = (sem0, sem1, sem2, sem3)
    NBUF = 4

    def row_copies(i, p):
        return (
            pltpu.make_async_copy(
                table_hbm.at[tok_v.at[i, pl.ds(0, CH0)]],
                buf_v.at[p, pl.ds(0, CH0), :],
                sems[p]),
            pltpu.make_async_copy(
                table_hbm.at[tok_v.at[i, pl.ds(CH0, CH1)]],
                buf_v.at[p, pl.ds(CH0, CH1), :],
                sems[p]),
        )

    def start_row(i, p):
        for c in row_copies(i, p):
            c.start()

    def wait_row(i, p):
        for c in row_copies(i, p):
            c.wait()

    UNROLL = 8

    def accum_row(i, p):
        def body(r, accs):
            accs = list(accs)
            for rr in range(UNROLL):
                row = r * UNROLL + rr
                for c in range(NCHUNK):
                    accs[c] = accs[c] + buf_v[p, row, pl.ds(c * L, L)]
            return tuple(accs)

        accs = tuple(jnp.zeros((L,), jnp.float32) for _ in range(NCHUNK))
        accs = lax.fori_loop(0, SEQ // UNROLL, body, accs)
        for c in range(NCHUNK):
            out_v[i, pl.ds(c * L, L)] = accs[c]

    for p in range(NBUF - 1):
        start_row(p, p)

    def step(k, carry):
        i0 = k * NBUF
        for b in range(NBUF):
            i = i0 + b

            @pl.when(i + NBUF - 1 < ROWS_PER_W)
            def _():
                start_row(i + NBUF - 1, (b + NBUF - 1) % NBUF)

            wait_row(i, b)
            accum_row(i, b)
        return carry

    lax.fori_loop(0, ROWS_PER_W // NBUF, step, 0)
    pltpu.sync_copy(out_v, out_hbm.at[pl.ds(base, ROWS_PER_W), :])


def _pack_body(in_ref, eye_ref, o_ref):
    x = in_ref[...]
    t = lax.dot_general(
        x, eye_ref[...], (((0,), (0,)), ((), ())),
        preferred_element_type=jnp.float32,
    )
    o_ref[...] = jnp.concatenate(
        [t[: PACK_C // 2, :], t[PACK_C // 2 :, :]], axis=1
    )


def _pack(tabT):
    return pl.pallas_call(
        _pack_body,
        grid=(PACK_NBLK,),
        in_specs=[
            pl.BlockSpec((EMBED, PACK_C), lambda j: (0, j)),
            pl.BlockSpec((EMBED, EMBED), lambda j: (0, 0)),
        ],
        out_specs=pl.BlockSpec((PACK_C // 2, 128), lambda j: (j, 0)),
        out_shape=jax.ShapeDtypeStruct((PACK_ROWS, 128), jnp.float32),
    )(tabT, jnp.eye(EMBED, dtype=jnp.float32))


def _mm_body(x_ref, w_ref, b_ref, o_ref):
    x = x_ref[...] * (1.0 / SEQ)
    o_ref[...] = (
        jnp.dot(x, w_ref[...], preferred_element_type=jnp.float32) + b_ref[...]
    )


def _matmul(pooled, W, b2d):
    return pl.pallas_call(
        _mm_body,
        out_shape=jax.ShapeDtypeStruct((BATCH, EMBED), jnp.float32),
    )(pooled, W, b2d)


def kernel(token_ids, embedding_table, W, b):
    tok = token_ids.astype(jnp.int32)
    packed = _pack(embedding_table.T)
    tab_rows = packed.reshape(2 * PACK_ROWS, EMBED)
    pooled = _pool_kernel(tok, tab_rows)
    return _matmul(pooled, W, b.reshape(1, EMBED))


# speedup vs baseline: 1.6336x; 1.3156x over previous
import functools

import jax
import jax.numpy as jnp
from jax import lax
from jax.experimental import pallas as pl
from jax.experimental.pallas import tpu as pltpu
from jax.experimental.pallas import tpu_sc as plsc

VOCAB = 1_000_000
EMBED = 64
BATCH = 4096
SEQ = 200

PACK_C = 4096
PACK_LOG = PACK_C.bit_length() - 1
PACK_NBLK = (VOCAB + PACK_C - 1) // PACK_C
PACK_ROWS = PACK_NBLK * PACK_C * EMBED // 128

NC = 2
NS = 16
L = 16
NW = NC * NS
ROWS_PER_W = BATCH // NW
CH0 = 104
CH1 = SEQ - CH0
NCHUNK = EMBED // L

_mesh = plsc.VectorSubcoreMesh(core_axis_name="c", subcore_axis_name="s")


@functools.partial(
    pl.kernel,
    out_type=jax.ShapeDtypeStruct((BATCH, EMBED), jnp.float32),
    mesh=_mesh,
    scratch_types=[
        pltpu.VMEM((ROWS_PER_W, 208), jnp.int32),
        pltpu.VMEM((4, SEQ, EMBED), jnp.float32),
        pltpu.VMEM((ROWS_PER_W, EMBED), jnp.float32),
        pltpu.SemaphoreType.DMA,
        pltpu.SemaphoreType.DMA,
        pltpu.SemaphoreType.DMA,
        pltpu.SemaphoreType.DMA,
    ],
    compiler_params=pltpu.CompilerParams(use_tc_tiling_on_sc=False),
)
def _pool_kernel(tok_hbm, table_hbm, out_hbm, tok_v, buf_v, out_v,
                 sem0, sem1, sem2, sem3):
    wid = lax.axis_index("s") * NC + lax.axis_index("c")
    base = wid * ROWS_PER_W
    pltpu.sync_copy(tok_hbm.at[pl.ds(base, ROWS_PER_W), :],
                    tok_v.at[:, pl.ds(0, SEQ)])

    def remap_body(i, carry):
        for c in range(208 // L):
            v = tok_v[i, pl.ds(c * L, L)]
            v = (
                ((v >> PACK_LOG) << PACK_LOG)
                + ((v & (PACK_C // 2 - 1)) << 1)
                + ((v >> (PACK_LOG - 1)) & 1)
            )
            tok_v[i, pl.ds(c * L, L)] = v
        return carry

    lax.fori_loop(0, ROWS_PER_W, remap_body, 0)

    sems = (sem0, sem1, sem2, sem3)
    NBUF = 4

    def row_copies(i, p):
        return (
            pltpu.make_async_copy(
                table_hbm.at[tok_v.at[i, pl.ds(0, CH0)]],
                buf_v.at[p, pl.ds(0, CH0), :],
                sems[p]),
            pltpu.make_async_copy(
                table_hbm.at[tok_v.at[i, pl.ds(CH0, CH1)]],
                buf_v.at[p, pl.ds(CH0, CH1), :],
                sems[p]),
        )

    def start_row(i, p):
        for c in row_copies(i, p):
            c.start()

    def wait_row(i, p):
        for c in row_copies(i, p):
            c.wait()

    UNROLL = 8

    def accum_row(i, p):
        def body(r, accs):
            accs = list(accs)
            for rr in range(UNROLL):
                row = r * UNROLL + rr
                for c in range(NCHUNK):
                    accs[c] = accs[c] + buf_v[p, row, pl.ds(c * L, L)]
            return tuple(accs)

        accs = tuple(jnp.zeros((L,), jnp.float32) for _ in range(NCHUNK))
        accs = lax.fori_loop(0, SEQ // UNROLL, body, accs)
        for c in range(NCHUNK):
            out_v[i, pl.ds(c * L, L)] = accs[c]

    for p in range(NBUF - 1):
        start_row(p, p)

    def step(k, carry):
        i0 = k * NBUF
        for b in range(NBUF):
            i = i0 + b

            @pl.when(i + NBUF - 1 < ROWS_PER_W)
            def _():
                start_row(i + NBUF - 1, (b + NBUF - 1) % NBUF)

            wait_row(i, b)
            accum_row(i, b)
        return carry

    lax.fori_loop(0, ROWS_PER_W // NBUF, step, 0)
    pltpu.sync_copy(out_v, out_hbm.at[pl.ds(base, ROWS_PER_W), :])


def _pack_body(in_ref, o_ref):
    x = in_ref[...]
    t = jnp.transpose(x, (1, 0))
    o_ref[...] = jnp.concatenate(
        [t[: PACK_C // 2, :], t[PACK_C // 2 :, :]], axis=1
    )


def _pack(tabT):
    return pl.pallas_call(
        _pack_body,
        grid=(PACK_NBLK,),
        in_specs=[pl.BlockSpec((EMBED, PACK_C), lambda j: (0, j))],
        out_specs=pl.BlockSpec((PACK_C // 2, 128), lambda j: (j, 0)),
        out_shape=jax.ShapeDtypeStruct((PACK_ROWS, 128), jnp.float32),
    )(tabT)


def _mm_body(x_ref, w_ref, b_ref, o_ref):
    x = x_ref[...] * (1.0 / SEQ)
    o_ref[...] = (
        jnp.dot(x, w_ref[...], preferred_element_type=jnp.float32) + b_ref[...]
    )


def _matmul(pooled, W, b2d):
    return pl.pallas_call(
        _mm_body,
        out_shape=jax.ShapeDtypeStruct((BATCH, EMBED), jnp.float32),
    )(pooled, W, b2d)


def kernel(token_ids, embedding_table, W, b):
    tok = token_ids.astype(jnp.int32)
    packed = _pack(embedding_table.T)
    tab_rows = packed.reshape(2 * PACK_ROWS, EMBED)
    pooled = _pool_kernel(tok, tab_rows)
    return _matmul(pooled, W, b.reshape(1, EMBED))


# speedup vs baseline: 1.9262x; 1.1791x over previous
import functools

import jax
import jax.numpy as jnp
from jax import lax
from jax.experimental import pallas as pl
from jax.experimental.pallas import tpu as pltpu
from jax.experimental.pallas import tpu_sc as plsc

VOCAB = 1_000_000
EMBED = 64
BATCH = 4096
SEQ = 200

PACK_C = 8192
PACK_LOG = PACK_C.bit_length() - 1
PACK_NBLK = (VOCAB + PACK_C - 1) // PACK_C
PACK_ROWS = PACK_NBLK * PACK_C * EMBED // 128

NC = 2
NS = 16
L = 16
NW = NC * NS
ROWS_PER_W = BATCH // NW
CH0 = 104
CH1 = SEQ - CH0
NCHUNK = EMBED // L

_mesh = plsc.VectorSubcoreMesh(core_axis_name="c", subcore_axis_name="s")


@functools.partial(
    pl.kernel,
    out_type=jax.ShapeDtypeStruct((BATCH, EMBED), jnp.float32),
    mesh=_mesh,
    scratch_types=[
        pltpu.VMEM((ROWS_PER_W, 208), jnp.int32),
        pltpu.VMEM((4, SEQ, EMBED), jnp.float32),
        pltpu.VMEM((ROWS_PER_W, EMBED), jnp.float32),
        pltpu.SemaphoreType.DMA,
        pltpu.SemaphoreType.DMA,
        pltpu.SemaphoreType.DMA,
        pltpu.SemaphoreType.DMA,
    ],
    compiler_params=pltpu.CompilerParams(use_tc_tiling_on_sc=False),
)
def _pool_kernel(tok_hbm, table_hbm, out_hbm, tok_v, buf_v, out_v,
                 sem0, sem1, sem2, sem3):
    wid = lax.axis_index("s") * NC + lax.axis_index("c")
    base = wid * ROWS_PER_W
    pltpu.sync_copy(tok_hbm.at[pl.ds(base, ROWS_PER_W), :],
                    tok_v.at[:, pl.ds(0, SEQ)])

    def remap_body(i, carry):
        for c in range(208 // L):
            v = tok_v[i, pl.ds(c * L, L)]
            v = (
                ((v >> PACK_LOG) << PACK_LOG)
                + ((v & (PACK_C // 2 - 1)) << 1)
                + ((v >> (PACK_LOG - 1)) & 1)
            )
            tok_v[i, pl.ds(c * L, L)] = v
        return carry

    lax.fori_loop(0, ROWS_PER_W, remap_body, 0)

    sems = (sem0, sem1, sem2, sem3)
    NBUF = 4

    def row_copies(i, p):
        return (
            pltpu.make_async_copy(
                table_hbm.at[tok_v.at[i, pl.ds(0, CH0)]],
                buf_v.at[p, pl.ds(0, CH0), :],
                sems[p]),
            pltpu.make_async_copy(
                table_hbm.at[tok_v.at[i, pl.ds(CH0, CH1)]],
                buf_v.at[p, pl.ds(CH0, CH1), :],
                sems[p]),
        )

    def start_row(i, p):
        for c in row_copies(i, p):
            c.start()

    def wait_row(i, p):
        for c in row_copies(i, p):
            c.wait()

    UNROLL = 8

    def accum_row(i, p):
        def body(r, accs):
            accs = list(accs)
            for rr in range(UNROLL):
                row = r * UNROLL + rr
                for c in range(NCHUNK):
                    accs[c] = accs[c] + buf_v[p, row, pl.ds(c * L, L)]
            return tuple(accs)

        accs = tuple(jnp.zeros((L,), jnp.float32) for _ in range(NCHUNK))
        accs = lax.fori_loop(0, SEQ // UNROLL, body, accs)
        for c in range(NCHUNK):
            out_v[i, pl.ds(c * L, L)] = accs[c]

    for p in range(NBUF - 1):
        start_row(p, p)

    def step(k, carry):
        i0 = k * NBUF
        for b in range(NBUF):
            i = i0 + b

            @pl.when(i + NBUF - 1 < ROWS_PER_W)
            def _():
                start_row(i + NBUF - 1, (b + NBUF - 1) % NBUF)

            wait_row(i, b)
            accum_row(i, b)
        return carry

    lax.fori_loop(0, ROWS_PER_W // NBUF, step, 0)
    pltpu.sync_copy(out_v, out_hbm.at[pl.ds(base, ROWS_PER_W), :])


def _pack_body(in_ref, o_ref):
    x = in_ref[...]
    t = jnp.transpose(x, (1, 0))
    o_ref[...] = jnp.concatenate(
        [t[: PACK_C // 2, :], t[PACK_C // 2 :, :]], axis=1
    )


def _pack(tabT):
    return pl.pallas_call(
        _pack_body,
        grid=(PACK_NBLK,),
        in_specs=[pl.BlockSpec((EMBED, PACK_C), lambda j: (0, j))],
        out_specs=pl.BlockSpec((PACK_C // 2, 128), lambda j: (j, 0)),
        out_shape=jax.ShapeDtypeStruct((PACK_ROWS, 128), jnp.float32),
    )(tabT)


def _mm_body(x_ref, w_ref, b_ref, o_ref):
    x = x_ref[...] * (1.0 / SEQ)
    o_ref[...] = (
        jnp.dot(x, w_ref[...], preferred_element_type=jnp.float32) + b_ref[...]
    )


def _matmul(pooled, W, b2d):
    return pl.pallas_call(
        _mm_body,
        out_shape=jax.ShapeDtypeStruct((BATCH, EMBED), jnp.float32),
    )(pooled, W, b2d)


def kernel(token_ids, embedding_table, W, b):
    tok = token_ids.astype(jnp.int32)
    packed = _pack(embedding_table.T)
    tab_rows = packed.reshape(2 * PACK_ROWS, EMBED)
    pooled = _pool_kernel(tok, tab_rows)
    return _matmul(pooled, W, b.reshape(1, EMBED))


# speedup vs baseline: 2.1244x; 1.1029x over previous
import functools

import jax
import jax.numpy as jnp
from jax import lax
from jax.experimental import pallas as pl
from jax.experimental.pallas import tpu as pltpu
from jax.experimental.pallas import tpu_sc as plsc

VOCAB = 1_000_000
EMBED = 64
BATCH = 4096
SEQ = 200

PACK_C = 16384
PACK_LOG = PACK_C.bit_length() - 1
PACK_NBLK = (VOCAB + PACK_C - 1) // PACK_C
PACK_ROWS = PACK_NBLK * PACK_C * EMBED // 128

NC = 2
NS = 16
L = 16
NW = NC * NS
ROWS_PER_W = BATCH // NW
CH0 = 104
CH1 = SEQ - CH0
NCHUNK = EMBED // L

_mesh = plsc.VectorSubcoreMesh(core_axis_name="c", subcore_axis_name="s")


@functools.partial(
    pl.kernel,
    out_type=jax.ShapeDtypeStruct((BATCH, EMBED), jnp.float32),
    mesh=_mesh,
    scratch_types=[
        pltpu.VMEM((ROWS_PER_W, 208), jnp.int32),
        pltpu.VMEM((4, SEQ, EMBED), jnp.float32),
        pltpu.VMEM((ROWS_PER_W, EMBED), jnp.float32),
        pltpu.SemaphoreType.DMA,
        pltpu.SemaphoreType.DMA,
        pltpu.SemaphoreType.DMA,
        pltpu.SemaphoreType.DMA,
    ],
    compiler_params=pltpu.CompilerParams(use_tc_tiling_on_sc=False),
)
def _pool_kernel(tok_hbm, table_hbm, out_hbm, tok_v, buf_v, out_v,
                 sem0, sem1, sem2, sem3):
    wid = lax.axis_index("s") * NC + lax.axis_index("c")
    base = wid * ROWS_PER_W
    pltpu.sync_copy(tok_hbm.at[pl.ds(base, ROWS_PER_W), :],
                    tok_v.at[:, pl.ds(0, SEQ)])

    def remap_body(i, carry):
        for c in range(208 // L):
            v = tok_v[i, pl.ds(c * L, L)]
            v = (
                ((v >> PACK_LOG) << PACK_LOG)
                + ((v & (PACK_C // 2 - 1)) << 1)
                + ((v >> (PACK_LOG - 1)) & 1)
            )
            tok_v[i, pl.ds(c * L, L)] = v
        return carry

    lax.fori_loop(0, ROWS_PER_W, remap_body, 0)

    sems = (sem0, sem1, sem2, sem3)
    NBUF = 4

    def row_copies(i, p):
        return (
            pltpu.make_async_copy(
                table_hbm.at[tok_v.at[i, pl.ds(0, CH0)]],
                buf_v.at[p, pl.ds(0, CH0), :],
                sems[p]),
            pltpu.make_async_copy(
                table_hbm.at[tok_v.at[i, pl.ds(CH0, CH1)]],
                buf_v.at[p, pl.ds(CH0, CH1), :],
                sems[p]),
        )

    def start_row(i, p):
        for c in row_copies(i, p):
            c.start()

    def wait_row(i, p):
        for c in row_copies(i, p):
            c.wait()

    UNROLL = 8

    def accum_row(i, p):
        def body(r, accs):
            accs = list(accs)
            for rr in range(UNROLL):
                row = r * UNROLL + rr
                for c in range(NCHUNK):
                    accs[c] = accs[c] + buf_v[p, row, pl.ds(c * L, L)]
            return tuple(accs)

        accs = tuple(jnp.zeros((L,), jnp.float32) for _ in range(NCHUNK))
        accs = lax.fori_loop(0, SEQ // UNROLL, body, accs)
        for c in range(NCHUNK):
            out_v[i, pl.ds(c * L, L)] = accs[c]

    for p in range(NBUF - 1):
        start_row(p, p)

    def step(k, carry):
        i0 = k * NBUF
        for b in range(NBUF):
            i = i0 + b

            @pl.when(i + NBUF - 1 < ROWS_PER_W)
            def _():
                start_row(i + NBUF - 1, (b + NBUF - 1) % NBUF)

            wait_row(i, b)
            accum_row(i, b)
        return carry

    lax.fori_loop(0, ROWS_PER_W // NBUF, step, 0)
    pltpu.sync_copy(out_v, out_hbm.at[pl.ds(base, ROWS_PER_W), :])


def _pack_body(in_ref, o_ref):
    x = in_ref[...]
    t = jnp.transpose(x, (1, 0))
    o_ref[...] = jnp.concatenate(
        [t[: PACK_C // 2, :], t[PACK_C // 2 :, :]], axis=1
    )


def _pack(tabT):
    return pl.pallas_call(
        _pack_body,
        grid=(PACK_NBLK,),
        in_specs=[pl.BlockSpec((EMBED, PACK_C), lambda j: (0, j))],
        out_specs=pl.BlockSpec((PACK_C // 2, 128), lambda j: (j, 0)),
        out_shape=jax.ShapeDtypeStruct((PACK_ROWS, 128), jnp.float32),
    )(tabT)


def _mm_body(x_ref, w_ref, b_ref, o_ref):
    x = x_ref[...] * (1.0 / SEQ)
    o_ref[...] = (
        jnp.dot(x, w_ref[...], preferred_element_type=jnp.float32) + b_ref[...]
    )


def _matmul(pooled, W, b2d):
    return pl.pallas_call(
        _mm_body,
        out_shape=jax.ShapeDtypeStruct((BATCH, EMBED), jnp.float32),
    )(pooled, W, b2d)


def kernel(token_ids, embedding_table, W, b):
    tok = token_ids.astype(jnp.int32)
    packed = _pack(embedding_table.T)
    tab_rows = packed.reshape(2 * PACK_ROWS, EMBED)
    pooled = _pool_kernel(tok, tab_rows)
    return _matmul(pooled, W, b.reshape(1, EMBED))


# speedup vs baseline: 2.2246x; 1.0472x over previous
import functools

import jax
import jax.numpy as jnp
from jax import lax
from jax.experimental import pallas as pl
from jax.experimental.pallas import tpu as pltpu
from jax.experimental.pallas import tpu_sc as plsc

VOCAB = 1_000_000
EMBED = 64
BATCH = 4096
SEQ = 200

PACK_C = 32768
PACK_LOG = PACK_C.bit_length() - 1
PACK_NBLK = (VOCAB + PACK_C - 1) // PACK_C
PACK_ROWS = PACK_NBLK * PACK_C * EMBED // 128

NC = 2
NS = 16
L = 16
NW = NC * NS
ROWS_PER_W = BATCH // NW
CH0 = 104
CH1 = SEQ - CH0
NCHUNK = EMBED // L

_mesh = plsc.VectorSubcoreMesh(core_axis_name="c", subcore_axis_name="s")


@functools.partial(
    pl.kernel,
    out_type=jax.ShapeDtypeStruct((BATCH, EMBED), jnp.float32),
    mesh=_mesh,
    scratch_types=[
        pltpu.VMEM((ROWS_PER_W, 208), jnp.int32),
        pltpu.VMEM((4, SEQ, EMBED), jnp.float32),
        pltpu.VMEM((ROWS_PER_W, EMBED), jnp.float32),
        pltpu.SemaphoreType.DMA,
        pltpu.SemaphoreType.DMA,
        pltpu.SemaphoreType.DMA,
        pltpu.SemaphoreType.DMA,
    ],
    compiler_params=pltpu.CompilerParams(use_tc_tiling_on_sc=False),
)
def _pool_kernel(tok_hbm, table_hbm, out_hbm, tok_v, buf_v, out_v,
                 sem0, sem1, sem2, sem3):
    wid = lax.axis_index("s") * NC + lax.axis_index("c")
    base = wid * ROWS_PER_W
    pltpu.sync_copy(tok_hbm.at[pl.ds(base, ROWS_PER_W), :],
                    tok_v.at[:, pl.ds(0, SEQ)])

    def remap_body(i, carry):
        for c in range(208 // L):
            v = tok_v[i, pl.ds(c * L, L)]
            v = (
                ((v >> PACK_LOG) << PACK_LOG)
                + ((v & (PACK_C // 2 - 1)) << 1)
                + ((v >> (PACK_LOG - 1)) & 1)
            )
            tok_v[i, pl.ds(c * L, L)] = v
        return carry

    lax.fori_loop(0, ROWS_PER_W, remap_body, 0)

    sems = (sem0, sem1, sem2, sem3)
    NBUF = 4

    def row_copies(i, p):
        return (
            pltpu.make_async_copy(
                table_hbm.at[tok_v.at[i, pl.ds(0, CH0)]],
                buf_v.at[p, pl.ds(0, CH0), :],
                sems[p]),
            pltpu.make_async_copy(
                table_hbm.at[tok_v.at[i, pl.ds(CH0, CH1)]],
                buf_v.at[p, pl.ds(CH0, CH1), :],
                sems[p]),
        )

    def start_row(i, p):
        for c in row_copies(i, p):
            c.start()

    def wait_row(i, p):
        for c in row_copies(i, p):
            c.wait()

    UNROLL = 8

    def accum_row(i, p):
        def body(r, accs):
            accs = list(accs)
            for rr in range(UNROLL):
                row = r * UNROLL + rr
                for c in range(NCHUNK):
                    accs[c] = accs[c] + buf_v[p, row, pl.ds(c * L, L)]
            return tuple(accs)

        accs = tuple(jnp.zeros((L,), jnp.float32) for _ in range(NCHUNK))
        accs = lax.fori_loop(0, SEQ // UNROLL, body, accs)
        for c in range(NCHUNK):
            out_v[i, pl.ds(c * L, L)] = accs[c]

    for p in range(NBUF - 1):
        start_row(p, p)

    def step(k, carry):
        i0 = k * NBUF
        for b in range(NBUF):
            i = i0 + b

            @pl.when(i + NBUF - 1 < ROWS_PER_W)
            def _():
                start_row(i + NBUF - 1, (b + NBUF - 1) % NBUF)

            wait_row(i, b)
            accum_row(i, b)
        return carry

    lax.fori_loop(0, ROWS_PER_W // NBUF, step, 0)
    pltpu.sync_copy(out_v, out_hbm.at[pl.ds(base, ROWS_PER_W), :])


def _pack_body(in_ref, o_ref):
    x = in_ref[...]
    t = jnp.transpose(x, (1, 0))
    o_ref[...] = jnp.concatenate(
        [t[: PACK_C // 2, :], t[PACK_C // 2 :, :]], axis=1
    )


def _pack(tabT):
    return pl.pallas_call(
        _pack_body,
        grid=(PACK_NBLK,),
        in_specs=[pl.BlockSpec((EMBED, PACK_C), lambda j: (0, j))],
        out_specs=pl.BlockSpec((PACK_C // 2, 128), lambda j: (j, 0)),
        out_shape=jax.ShapeDtypeStruct((PACK_ROWS, 128), jnp.float32),
    )(tabT)


def _mm_body(x_ref, w_ref, b_ref, o_ref):
    x = x_ref[...] * (1.0 / SEQ)
    o_ref[...] = (
        jnp.dot(x, w_ref[...], preferred_element_type=jnp.float32) + b_ref[...]
    )


def _matmul(pooled, W, b2d):
    return pl.pallas_call(
        _mm_body,
        out_shape=jax.ShapeDtypeStruct((BATCH, EMBED), jnp.float32),
    )(pooled, W, b2d)


def kernel(token_ids, embedding_table, W, b):
    tok = token_ids.astype(jnp.int32)
    packed = _pack(embedding_table.T)
    tab_rows = packed.reshape(2 * PACK_ROWS, EMBED)
    pooled = _pool_kernel(tok, tab_rows)
    return _matmul(pooled, W, b.reshape(1, EMBED))
